# baseline (speedup 1.0000x reference)
import functools

import jax
import jax.numpy as jnp
from jax import lax
from jax.experimental import pallas as pl
from jax.experimental.pallas import tpu as pltpu
from jax.experimental.pallas import tpu_sc as plsc

L = 1024
K = 64
D_MSA = 256
D_PAIR = 64
D_STATE = 32
D_NODE = 32
D_EDGE = 32
N_RBF = 36

TC1_R = 256
TC2_R = 32
TC2_E = TC2_R * K

_NC, _NS = 2, 16
_NW = _NC * _NS
_CHUNK = 128
_ROWS_PER_W = (L * K) // _NW // _CHUNK
_FIRE = 2
_DG = 2 * D_PAIR


def _ln(x, g, b):
    m = jnp.mean(x, axis=-1, keepdims=True)
    v = jnp.mean((x - m) ** 2, axis=-1, keepdims=True)
    return (x - m) / jnp.sqrt(v + 1e-5) * g + b


def _norm(x):
    m = jnp.mean(x, axis=-1, keepdims=True)
    v = jnp.mean(x * x, axis=-1, keepdims=True) - m * m
    return (x - m) * (1.0 / jnp.sqrt(v + 1e-5))


def _tc1_body(msa_r, seq_r, st_r, xyzr_r, idxf_r, cat_r,
              lmg_r, lmb_r, lsg_r, lsb_r, wx_r, bx_r, lng_r, lnb_r,
              G_o, nbr_o, nbrd_o):
    i = pl.program_id(0)
    msan = _ln(msa_r[...], lmg_r[...], lmb_r[...])
    stn = _ln(st_r[...], lsg_r[...], lsb_r[...])
    nodecat = jnp.concatenate([msan, seq_r[...], stn], axis=1)
    node = jnp.dot(nodecat, wx_r[...], preferred_element_type=jnp.float32) + bx_r[...]
    h = _ln(node, lng_r[...], lnb_r[...])
    xyzb = xyzr_r[...]
    ca_blk = xyzb[:, 3:6]
    v = xyzb - jnp.concatenate([ca_blk, ca_blk, ca_blk], axis=1)
    pad = jnp.zeros((TC1_R, 128 - 45), jnp.float32)
    G_o[...] = jnp.concatenate([h, ca_blk, v, idxf_r[...], pad], axis=1)

    dx = xyzb[:, 3:4] - cat_r[0:1, :]
    dy = xyzb[:, 4:5] - cat_r[1:2, :]
    dz = xyzb[:, 5:6] - cat_r[2:3, :]
    dist = jnp.sqrt(dx * dx + dy * dy + dz * dz + 1e-8)
    rows = lax.broadcasted_iota(jnp.int32, (TC1_R, L), 0) + i * TC1_R
    cols = lax.broadcasted_iota(jnp.int32, (TC1_R, L), 1)
    dist = jnp.where(rows == cols, dist + 1e9, dist)

    nbrs, dvals = [], []
    for _ in range(K):
        m = jnp.min(dist, axis=1, keepdims=True)
        colc = jnp.where(dist == m, cols, jnp.int32(2 * L))
        c = jnp.min(colc, axis=1, keepdims=True)
        nbrs.append(c)
        dvals.append(m)
        dist = jnp.where(cols == c, jnp.float32(3e38), dist)
    nbr = jnp.concatenate(nbrs, axis=1)
    nbr_o[...] = nbr
    nbrd_o[...] = jnp.concatenate(dvals, axis=1)


def _run_tc1(msa, seq1hot, state, xyzr, idxf, ca_t, p):
    full = lambda shp: pl.BlockSpec(shp, lambda i: (0, 0))
    blk = lambda shp: pl.BlockSpec(shp, lambda i: (i, 0))
    return pl.pallas_call(
        _tc1_body,
        grid=(L // TC1_R,),
        in_specs=[
            blk((TC1_R, D_MSA)),
            blk((TC1_R, 21)),
            blk((TC1_R, D_STATE)),
            blk((TC1_R, 9)),
            blk((TC1_R, 1)),
            full((3, L)),
            full((1, D_MSA)), full((1, D_MSA)),
            full((1, D_STATE)), full((1, D_STATE)),
            full((D_MSA + 21 + D_STATE, D_NODE)), full((1, D_NODE)),
            full((1, D_NODE)), full((1, D_NODE)),
        ],
        out_specs=[
            blk((TC1_R, 128)),
            blk((TC1_R, K)),
            blk((TC1_R, K)),
        ],
        out_shape=[
            jax.ShapeDtypeStruct((L, 128), jnp.float32),
            jax.ShapeDtypeStruct((L, K), jnp.int32),
            jax.ShapeDtypeStruct((L, K), jnp.float32),
        ],
    )(msa, seq1hot, state, xyzr, idxf, ca_t,
      p['ln_msa_g'], p['ln_msa_b'], p['ln_state_g'], p['ln_state_b'],
      p['W_x'], p['b_x'], p['ln_node_g'], p['ln_node_b'])


def _sc_gather_body(rows_per_w, table_hbm, idx_hbm, out_hbm,
                    idx_v, rows_v, sem, wsem):
    wid = lax.axis_index("s") * _NC + lax.axis_index("c")
    base = wid * rows_per_w
    nrounds = rows_per_w // _FIRE
    pltpu.sync_copy(idx_hbm.at[pl.ds(base, rows_per_w)], idx_v)

    def fire(r):
        buf = r % 2
        return [
            pltpu.async_copy(
                table_hbm.at[idx_v.at[r * _FIRE + j]],
                rows_v.at[buf].at[j], sem)
            for j in range(_FIRE)
        ]
    gath = fire(0)
    wr = [None] * nrounds
    for r in range(nrounds):
        buf = r % 2
        if r + 1 < nrounds:
            if r >= 1:
                wr[r - 1].wait()
            nxt = fire(r + 1)
        for cp in gath:
            cp.wait()
        wr[r] = pltpu.async_copy(
            rows_v.at[buf], out_hbm.at[pl.ds(base + r * _FIRE, _FIRE)], wsem)
        if r + 1 < nrounds:
            gath = nxt
    wr[nrounds - 2].wait()
    wr[nrounds - 1].wait()


@functools.cache
def _make_sc_gather(nedges):
    rows_per_w = nedges // _NW // _CHUNK
    return functools.partial(
        pl.kernel,
        mesh=plsc.VectorSubcoreMesh(
            core_axis_name="c", subcore_axis_name="s", num_cores=_NC),
        out_type=jax.ShapeDtypeStruct((nedges // _CHUNK, _CHUNK, _DG),
                                      jnp.float32),
        scratch_types=[
            pltpu.VMEM((rows_per_w, _CHUNK), jnp.int32),
            pltpu.VMEM((2, _FIRE, _CHUNK, _DG), jnp.float32),
            pltpu.SemaphoreType.DMA,
            pltpu.SemaphoreType.DMA,
        ],
    )(functools.partial(_sc_gather_body, rows_per_w))


def _tc2_body(row0, pairt_r, gsrc_r, nbr_r, nbrd_r, own_r, cen_r,
              we1_r, be1_r, we2_r, be2_r,
              wm_r, bm_r, pa_r, wh_r, bh_r, wrep_r,
              wo_r, bo_r, wl_r, bl_r,
              xyz_o, lddt_o):
    i = pl.program_id(0)
    nbrc = nbr_r[...]
    onehot = (nbrc == lax.broadcasted_iota(jnp.int32, (TC2_E, L), 1)
              ).astype(jnp.float32)
    eraw = lax.dot_general(
        onehot.reshape(TC2_R, K, L), pairt_r[...],
        (((2,), (2,)), ((0,), (0,))),
        preferred_element_type=jnp.float32).reshape(TC2_E, D_PAIR)

    gath = gsrc_r[...]
    h_src = gath[:, 0:32]
    ca_src = gath[:, 32:35]
    v_src = gath[:, 35:44]
    idx_src = gath[:, 44:45]

    own = own_r[...]
    h16 = own[:, 0:32]
    ca16 = own[:, 32:35]
    v16 = own[:, 35:44]
    idx16 = own[:, 44:45]

    def rep(x):
        f = x.shape[1]
        return jnp.broadcast_to(x[:, None, :], (TC2_R, K, f)).reshape(TC2_E, f)

    h_dst = rep(h16)
    ca_dst = rep(ca16)
    idx_dst = rep(idx16)
    relpos = ca_src - ca_dst

    z1 = jnp.dot(_norm(eraw), we1_r[...],
                 preferred_element_type=jnp.float32) + be1_r[...]
    d = nbrd_r[...]
    sigma = jnp.float32((22.0 - 2.0) / N_RBF)
    cen = cen_r[...]
    k2 = (cen / sigma) ** 2
    a0 = -((d / sigma) ** 2)
    a1 = d * (2.0 / (sigma * sigma))
    rbf = jnp.exp(a1 * cen - k2 + a0)
    seqsep = jnp.clip(idx_src - idx_dst, -32.0, 32.0) / 32.0
    e2in = jnp.concatenate([_norm(z1), rbf, seqsep], axis=1)
    z2 = jnp.dot(e2in, we2_r[...],
                 preferred_element_type=jnp.float32) + be2_r[...]

    msgin = jnp.concatenate([h_dst, h_src, _norm(z2)], axis=1)
    mz = jnp.dot(msgin, wm_r[...], preferred_element_type=jnp.float32) + bm_r[...]
    a = pa_r[0, 0]
    msg = jnp.where(mz > 0, mz, a * mz)

    ab = jnp.dot(msg, wrep_r[...], preferred_element_type=jnp.float32)
    prod = (ab[:, 0:9] * jnp.concatenate([relpos, relpos, relpos], axis=1)
            + ab[:, 9:18] * v_src)
    seg = (lax.broadcasted_iota(jnp.int32, (TC2_R, TC2_E), 1) // K
           == lax.broadcasted_iota(jnp.int32, (TC2_R, TC2_E), 0)
           ).astype(jnp.float32)
    red = jnp.dot(seg, jnp.concatenate([msg, prod], axis=1),
                  preferred_element_type=jnp.float32)
    msg_sum = red[:, 0:D_NODE]
    acc = red[:, D_NODE:D_NODE + 9]
    v_new = v16 + acc / 64.0
    ca_new = ca16 + v_new[:, 3:6]
    grow = lax.broadcasted_iota(jnp.int32, (TC2_R, 1), 0) + i * TC2_R + row0
    ca_set = jnp.where(grow == 0, 0.0, ca_new)
    xyz_o[...] = v_new + jnp.concatenate([ca_set, ca_set, ca_set], axis=1)

    h_new = h16 + jnp.dot(msg_sum, wh_r[...],
                          preferred_element_type=jnp.float32) + bh_r[...]
    shift0 = jnp.dot(h_new, wo_r[...],
                     preferred_element_type=jnp.float32) + bo_r[...]
    z = jnp.dot(_norm(shift0), wl_r[...],
                preferred_element_type=jnp.float32) + bl_r[...]
    lddt_o[...] = jax.nn.sigmoid(z)


def _run_tc2(pairt, gsrc, nbr_col, nbrd_col, G, centers, p, half, nhalf):
    HB = L // nhalf
    base = half * (HB // TC2_R)
    full = lambda shp: pl.BlockSpec(shp, lambda i: (0, 0))
    blk = lambda shp: pl.BlockSpec(shp, lambda i: (i, 0))
    off = lambda shp: pl.BlockSpec(shp, lambda i: (i + base, 0))
    we1g = p['W_e1'] * p['ln_pair_g'].reshape(-1, 1)
    be1f = p['b_e1'] + p['ln_pair_b'] @ p['W_e1']
    we2g = jnp.concatenate(
        [p['W_e2'][:D_EDGE] * p['ln_e1_g'].reshape(-1, 1),
         p['W_e2'][D_EDGE:]], axis=0)
    be2f = p['b_e2'] + p['ln_e1_b'] @ p['W_e2'][:D_EDGE]
    wmg = jnp.concatenate(
        [p['W_msg'][:2 * D_NODE],
         p['W_msg'][2 * D_NODE:] * p['ln_e2_g'].reshape(-1, 1)], axis=0)
    bmf = p['b_msg'] + p['ln_e2_b'] @ p['W_msg'][2 * D_NODE:]
    wlg = p['W_lddt'] * p['ln_state_g'].reshape(-1, 1)
    blf = p['b_lddt'] + p['ln_state_b'] @ p['W_lddt']
    rep9 = lambda w: jnp.concatenate([w[:, c:c + 1] for c in (0, 0, 0, 1, 1, 1,
                                                              2, 2, 2)], axis=1)
    wrep = jnp.concatenate([rep9(p['W_rel']), rep9(p['W_nb'])], axis=1)
    return pl.pallas_call(
        functools.partial(_tc2_body, half * HB),
        grid=(HB // TC2_R,),
        in_specs=[
            pl.BlockSpec((TC2_R, D_PAIR, L), lambda i: (i + base, 0, 0)),
            blk((TC2_E, _DG)),
            off((TC2_E, 1)),
            off((TC2_E, 1)),
            off((TC2_R, 128)),
            full((1, N_RBF)),
            full((D_PAIR, D_EDGE)), full((1, D_EDGE)),
            full((D_EDGE + N_RBF + 1, D_EDGE)), full((1, D_EDGE)),
            full((2 * D_NODE + D_EDGE, D_NODE)), full((1, D_NODE)),
            full((1, 1)),
            full((D_NODE, D_NODE)), full((1, D_NODE)),
            full((D_NODE, 18)),
            full((D_NODE, D_STATE)), full((1, D_STATE)),
            full((D_STATE, 1)), full((1, 1)),
        ],
        out_specs=[
            blk((TC2_R, 9)),
            blk((TC2_R, 1)),
        ],
        out_shape=[
            jax.ShapeDtypeStruct((HB, 9), jnp.float32),
            jax.ShapeDtypeStruct((HB, 1), jnp.float32),
        ],
    )(pairt, gsrc, nbr_col, nbrd_col, G, centers,
      we1g, be1f, we2g, be2f, wmg, bmf, p['prelu_a'],
      p['W_h'], p['b_h'], wrep,
      p['W_out0'], p['b_out0'], wlg, blf)


def _node_gather(G, nbr_flat):
    n = nbr_flat.shape[0]
    idx2d = nbr_flat.reshape(n // _CHUNK, _CHUNK)
    out = _make_sc_gather(n)(G, idx2d)
    return out.reshape(n, _DG)


def kernel(xyz, state, msa, pair, seq1hot, idx, CA_atom_index, top_k, params):
    del CA_atom_index, top_k
    p = {k: jnp.asarray(v) for k, v in params.items()}
    for k in list(p):
        if p[k].ndim == 1:
            p[k] = p[k].reshape(1, -1)
        elif p[k].ndim == 0:
            p[k] = p[k].reshape(1, 1)

    xyzr = xyz[0].reshape(L, 9)
    ca_t = xyz[0, :, 1, :].T
    idxf = idx[0].astype(jnp.float32).reshape(L, 1)
    centers = jnp.linspace(2.0, 22.0, N_RBF, dtype=jnp.float32).reshape(1, -1)

    G, nbr, nbrd = _run_tc1(
        msa[0], seq1hot[0], state[0], xyzr, idxf, ca_t, p)

    pairt = jnp.transpose(pair, (0, 1, 3, 2)).reshape(L, D_PAIR, L)
    nbr_col = nbr.reshape(L * K, 1)
    nbrd_col = nbrd.reshape(L * K, 1)
    HB = L // 2
    gsrc_a = _node_gather(G, nbr[:HB].reshape(HB * K))
    gsrc_b = _node_gather(G, nbr[HB:].reshape(HB * K))
    xyz_a, lddt_a = _run_tc2(pairt, gsrc_a, nbr_col, nbrd_col, G, centers,
                             p, 0, 2)
    xyz_b, lddt_b = _run_tc2(pairt, gsrc_b, nbr_col, nbrd_col, G, centers,
                             p, 1, 2)
    xyz_o = jnp.concatenate([xyz_a, xyz_b], axis=0)
    lddt_o = jnp.concatenate([lddt_a, lddt_b], axis=0)

    return xyz_o.reshape(L * 3, 3), lddt_o.reshape(1, L, 1)


# speedup vs baseline: 1.0459x; 1.0459x over previous
import functools

import jax
import jax.numpy as jnp
from jax import lax
from jax.experimental import pallas as pl
from jax.experimental.pallas import tpu as pltpu
from jax.experimental.pallas import tpu_sc as plsc

L = 1024
K = 64
D_MSA = 256
D_PAIR = 64
D_STATE = 32
D_NODE = 32
D_EDGE = 32
N_RBF = 36

TC1_R = 256
TC2_R = 64
TC2_E = TC2_R * K

_NC, _NS = 2, 16
_NW = _NC * _NS
_CHUNK = 128
_ROWS_PER_W = (L * K) // _NW // _CHUNK
_FIRE = 2
_DG = 128


def _ln(x, g, b):
    m = jnp.mean(x, axis=-1, keepdims=True)
    v = jnp.mean((x - m) ** 2, axis=-1, keepdims=True)
    return (x - m) / jnp.sqrt(v + 1e-5) * g + b


def _norm(x):
    m = jnp.mean(x, axis=-1, keepdims=True)
    v = jnp.mean(x * x, axis=-1, keepdims=True) - m * m
    return (x - m) * (1.0 / jnp.sqrt(v + 1e-5))


def _tc1_body(msa_r, seq_r, st_r, xyzr_r, idxf_r, cat_r,
              lmg_r, lmb_r, lsg_r, lsb_r, wx_r, bx_r, lng_r, lnb_r,
              G_o, nbr_o, nbrd_o):
    i = pl.program_id(0)
    msan = _ln(msa_r[...], lmg_r[...], lmb_r[...])
    stn = _ln(st_r[...], lsg_r[...], lsb_r[...])
    nodecat = jnp.concatenate([msan, seq_r[...], stn], axis=1)
    node = jnp.dot(nodecat, wx_r[...], preferred_element_type=jnp.float32) + bx_r[...]
    h = _ln(node, lng_r[...], lnb_r[...])
    xyzb = xyzr_r[...]
    ca_blk = xyzb[:, 3:6]
    v = xyzb - jnp.concatenate([ca_blk, ca_blk, ca_blk], axis=1)
    pad = jnp.zeros((TC1_R, 128 - 45), jnp.float32)
    G_o[...] = jnp.concatenate([h, ca_blk, v, idxf_r[...], pad], axis=1)

    dx = xyzb[:, 3:4] - cat_r[0:1, :]
    dy = xyzb[:, 4:5] - cat_r[1:2, :]
    dz = xyzb[:, 5:6] - cat_r[2:3, :]
    dist = jnp.sqrt(dx * dx + dy * dy + dz * dz + 1e-8)
    rows = lax.broadcasted_iota(jnp.int32, (TC1_R, L), 0) + i * TC1_R
    cols = lax.broadcasted_iota(jnp.int32, (TC1_R, L), 1)
    dist = jnp.where(rows == cols, dist + 1e9, dist)

    nbrs, dvals = [], []
    for _ in range(K):
        m = jnp.min(dist, axis=1, keepdims=True)
        colc = jnp.where(dist == m, cols, jnp.int32(2 * L))
        c = jnp.min(colc, axis=1, keepdims=True)
        nbrs.append(c)
        dvals.append(m)
        dist = jnp.where(cols == c, jnp.float32(3e38), dist)
    nbr = jnp.concatenate(nbrs, axis=1)
    nbr_o[...] = nbr
    nbrd_o[...] = jnp.concatenate(dvals, axis=1)


def _run_tc1(msa, seq1hot, state, xyzr, idxf, ca_t, p):
    full = lambda shp: pl.BlockSpec(shp, lambda i: (0, 0))
    blk = lambda shp: pl.BlockSpec(shp, lambda i: (i, 0))
    return pl.pallas_call(
        _tc1_body,
        grid=(L // TC1_R,),
        in_specs=[
            blk((TC1_R, D_MSA)),
            blk((TC1_R, 21)),
            blk((TC1_R, D_STATE)),
            blk((TC1_R, 9)),
            blk((TC1_R, 1)),
            full((3, L)),
            full((1, D_MSA)), full((1, D_MSA)),
            full((1, D_STATE)), full((1, D_STATE)),
            full((D_MSA + 21 + D_STATE, D_NODE)), full((1, D_NODE)),
            full((1, D_NODE)), full((1, D_NODE)),
        ],
        out_specs=[
            blk((TC1_R, 128)),
            blk((TC1_R, K)),
            blk((TC1_R, K)),
        ],
        out_shape=[
            jax.ShapeDtypeStruct((L, 128), jnp.float32),
            jax.ShapeDtypeStruct((L, K), jnp.int32),
            jax.ShapeDtypeStruct((L, K), jnp.float32),
        ],
    )(msa, seq1hot, state, xyzr, idxf, ca_t,
      p['ln_msa_g'], p['ln_msa_b'], p['ln_state_g'], p['ln_state_b'],
      p['W_x'], p['b_x'], p['ln_node_g'], p['ln_node_b'])


def _sc_gather_body(table_hbm, idx_hbm, out_hbm, idx_v, rows_v, sem, wsem):
    wid = lax.axis_index("s") * _NC + lax.axis_index("c")
    base = wid * _ROWS_PER_W
    nrounds = _ROWS_PER_W // _FIRE
    pltpu.sync_copy(idx_hbm.at[pl.ds(base, _ROWS_PER_W)], idx_v)

    def fire(r):
        buf = r % 2
        return [
            pltpu.async_copy(
                table_hbm.at[idx_v.at[r * _FIRE + j]],
                rows_v.at[buf].at[j], sem)
            for j in range(_FIRE)
        ]
    gath = fire(0)
    wr = [None] * nrounds
    for r in range(nrounds):
        buf = r % 2
        if r + 1 < nrounds:
            if r >= 1:
                wr[r - 1].wait()
            nxt = fire(r + 1)
        for cp in gath:
            cp.wait()
        wr[r] = pltpu.async_copy(
            rows_v.at[buf], out_hbm.at[pl.ds(base + r * _FIRE, _FIRE)], wsem)
        if r + 1 < nrounds:
            gath = nxt
    wr[nrounds - 2].wait()
    wr[nrounds - 1].wait()


@functools.cache
def _make_sc_gather():
    return functools.partial(
        pl.kernel,
        mesh=plsc.VectorSubcoreMesh(
            core_axis_name="c", subcore_axis_name="s", num_cores=_NC),
        out_type=jax.ShapeDtypeStruct((L * K // _CHUNK, _CHUNK, _DG),
                                      jnp.float32),
        scratch_types=[
            pltpu.VMEM((_ROWS_PER_W, _CHUNK), jnp.int32),
            pltpu.VMEM((2, _FIRE, _CHUNK, _DG), jnp.float32),
            pltpu.SemaphoreType.DMA,
            pltpu.SemaphoreType.DMA,
        ],
    )(_sc_gather_body)


def _tc2_body(pairt_r, gsrc_r, nbr_r, nbrd_r, own_r, cen_r,
              we1_r, be1_r, we2_r, be2_r,
              wm_r, bm_r, pa_r, wh_r, bh_r, wrep_r,
              wo_r, bo_r, wl_r, bl_r,
              xyz_o, lddt_o):
    i = pl.program_id(0)
    nbrc = nbr_r[...]
    onehot = (nbrc == lax.broadcasted_iota(jnp.int32, (TC2_E, L), 1)
              ).astype(jnp.float32)
    eraw = lax.dot_general(
        onehot.reshape(TC2_R, K, L), pairt_r[...],
        (((2,), (2,)), ((0,), (0,))),
        preferred_element_type=jnp.float32).reshape(TC2_E, D_PAIR)

    gath = gsrc_r[...]
    h_src = gath[:, 0:32]
    ca_src = gath[:, 32:35]
    v_src = gath[:, 35:44]
    idx_src = gath[:, 44:45]

    own = own_r[...]
    h16 = own[:, 0:32]
    ca16 = own[:, 32:35]
    v16 = own[:, 35:44]
    idx16 = own[:, 44:45]

    def rep(x):
        f = x.shape[1]
        return jnp.broadcast_to(x[:, None, :], (TC2_R, K, f)).reshape(TC2_E, f)

    h_dst = rep(h16)
    ca_dst = rep(ca16)
    idx_dst = rep(idx16)
    relpos = ca_src - ca_dst

    z1 = jnp.dot(_norm(eraw), we1_r[...],
                 preferred_element_type=jnp.float32) + be1_r[...]
    d = nbrd_r[...]
    sigma = jnp.float32((22.0 - 2.0) / N_RBF)
    cen = cen_r[...]
    k2 = (cen / sigma) ** 2
    a0 = -((d / sigma) ** 2)
    a1 = d * (2.0 / (sigma * sigma))
    rbf = jnp.exp(a1 * cen - k2 + a0)
    seqsep = jnp.clip(idx_src - idx_dst, -32.0, 32.0) / 32.0
    e2in = jnp.concatenate([_norm(z1), rbf, seqsep], axis=1)
    z2 = jnp.dot(e2in, we2_r[...],
                 preferred_element_type=jnp.float32) + be2_r[...]

    msgin = jnp.concatenate([h_dst, h_src, _norm(z2)], axis=1)
    mz = jnp.dot(msgin, wm_r[...], preferred_element_type=jnp.float32) + bm_r[...]
    a = pa_r[0, 0]
    msg = jnp.where(mz > 0, mz, a * mz)

    ab = jnp.dot(msg, wrep_r[...], preferred_element_type=jnp.float32)
    prod = (ab[:, 0:9] * jnp.concatenate([relpos, relpos, relpos], axis=1)
            + ab[:, 9:18] * v_src)
    seg = (lax.broadcasted_iota(jnp.int32, (TC2_R, TC2_E), 1) // K
           == lax.broadcasted_iota(jnp.int32, (TC2_R, TC2_E), 0)
           ).astype(jnp.float32)
    red = jnp.dot(seg, jnp.concatenate([msg, prod], axis=1),
                  preferred_element_type=jnp.float32)
    msg_sum = red[:, 0:D_NODE]
    acc = red[:, D_NODE:D_NODE + 9]
    v_new = v16 + acc / 64.0
    ca_new = ca16 + v_new[:, 3:6]
    grow = lax.broadcasted_iota(jnp.int32, (TC2_R, 1), 0) + i * TC2_R
    ca_set = jnp.where(grow == 0, 0.0, ca_new)
    xyz_o[...] = v_new + jnp.concatenate([ca_set, ca_set, ca_set], axis=1)

    h_new = h16 + jnp.dot(msg_sum, wh_r[...],
                          preferred_element_type=jnp.float32) + bh_r[...]
    shift0 = jnp.dot(h_new, wo_r[...],
                     preferred_element_type=jnp.float32) + bo_r[...]
    z = jnp.dot(_norm(shift0), wl_r[...],
                preferred_element_type=jnp.float32) + bl_r[...]
    lddt_o[...] = jax.nn.sigmoid(z)


def _run_tc2(pairt, gsrc, nbr_col, nbrd_col, G, centers, p):
    full = lambda shp: pl.BlockSpec(shp, lambda i: (0, 0))
    blk = lambda shp: pl.BlockSpec(shp, lambda i: (i, 0))
    we1g = p['W_e1'] * p['ln_pair_g'].reshape(-1, 1)
    be1f = p['b_e1'] + p['ln_pair_b'] @ p['W_e1']
    we2g = jnp.concatenate(
        [p['W_e2'][:D_EDGE] * p['ln_e1_g'].reshape(-1, 1),
         p['W_e2'][D_EDGE:]], axis=0)
    be2f = p['b_e2'] + p['ln_e1_b'] @ p['W_e2'][:D_EDGE]
    wmg = jnp.concatenate(
        [p['W_msg'][:2 * D_NODE],
         p['W_msg'][2 * D_NODE:] * p['ln_e2_g'].reshape(-1, 1)], axis=0)
    bmf = p['b_msg'] + p['ln_e2_b'] @ p['W_msg'][2 * D_NODE:]
    wlg = p['W_lddt'] * p['ln_state_g'].reshape(-1, 1)
    blf = p['b_lddt'] + p['ln_state_b'] @ p['W_lddt']
    rep9 = lambda w: jnp.concatenate([w[:, c:c + 1] for c in (0, 0, 0, 1, 1, 1,
                                                              2, 2, 2)], axis=1)
    wrep = jnp.concatenate([rep9(p['W_rel']), rep9(p['W_nb'])], axis=1)
    return pl.pallas_call(
        _tc2_body,
        grid=(L // TC2_R,),
        in_specs=[
            pl.BlockSpec((TC2_R, D_PAIR, L), lambda i: (i, 0, 0)),
            blk((TC2_E, _DG)),
            blk((TC2_E, 1)),
            blk((TC2_E, 1)),
            blk((TC2_R, 128)),
            full((1, N_RBF)),
            full((D_PAIR, D_EDGE)), full((1, D_EDGE)),
            full((D_EDGE + N_RBF + 1, D_EDGE)), full((1, D_EDGE)),
            full((2 * D_NODE + D_EDGE, D_NODE)), full((1, D_NODE)),
            full((1, 1)),
            full((D_NODE, D_NODE)), full((1, D_NODE)),
            full((D_NODE, 18)),
            full((D_NODE, D_STATE)), full((1, D_STATE)),
            full((D_STATE, 1)), full((1, 1)),
        ],
        out_specs=[
            blk((TC2_R, 9)),
            blk((TC2_R, 1)),
        ],
        out_shape=[
            jax.ShapeDtypeStruct((L, 9), jnp.float32),
            jax.ShapeDtypeStruct((L, 1), jnp.float32),
        ],
    )(pairt, gsrc, nbr_col, nbrd_col, G, centers,
      we1g, be1f, we2g, be2f, wmg, bmf, p['prelu_a'],
      p['W_h'], p['b_h'], wrep,
      p['W_out0'], p['b_out0'], wlg, blf)


def _node_gather(G, nbr_flat):
    idx2d = nbr_flat.reshape(L * K // _CHUNK, _CHUNK)
    out = _make_sc_gather()(G, idx2d)
    return out.reshape(L * K, _DG)


def kernel(xyz, state, msa, pair, seq1hot, idx, CA_atom_index, top_k, params):
    del CA_atom_index, top_k
    p = {k: jnp.asarray(v) for k, v in params.items()}
    for k in list(p):
        if p[k].ndim == 1:
            p[k] = p[k].reshape(1, -1)
        elif p[k].ndim == 0:
            p[k] = p[k].reshape(1, 1)

    xyzr = xyz[0].reshape(L, 9)
    ca_t = xyz[0, :, 1, :].T
    idxf = idx[0].astype(jnp.float32).reshape(L, 1)
    centers = jnp.linspace(2.0, 22.0, N_RBF, dtype=jnp.float32).reshape(1, -1)

    G, nbr, nbrd = _run_tc1(
        msa[0], seq1hot[0], state[0], xyzr, idxf, ca_t, p)

    gsrc = _node_gather(G, nbr.reshape(L * K))

    pairt = jnp.transpose(pair, (0, 1, 3, 2)).reshape(L, D_PAIR, L)
    nbr_col = nbr.reshape(L * K, 1)
    nbrd_col = nbrd.reshape(L * K, 1)
    xyz_o, lddt_o = _run_tc2(pairt, gsrc, nbr_col, nbrd_col, G, centers, p)

    return xyz_o.reshape(L * 3, 3), lddt_o.reshape(1, L, 1)
